# Initial kernel scaffold; baseline (speedup 1.0000x reference)
import jax
import jax.numpy as jnp
from jax.experimental import pallas as pl


def kernel(x, edge_index, W_f, W_b, bias):
    raise NotImplementedError("write your pallas kernel here")


# speedup vs baseline: 19.9676x; 19.9676x over previous
import functools

import jax
import jax.numpy as jnp
from jax import lax
from jax.experimental import pallas as pl
from jax.experimental.pallas import tpu as pltpu
from jax.experimental.pallas import tpu_sc as plsc

N = 10000
NP = 10240
E = 320000
D = 128
H = 64
DUMMY = 10200

NC = 2
NS = 16
C = 80
STRIPE = NP // NS

_mesh = plsc.VectorSubcoreMesh(core_axis_name="c", subcore_axis_name="s")

_f32 = jnp.float32
_i32 = jnp.int32


EPT_CNT = E // (NC * NS)
NCH_CNT = EPT_CNT // C


@functools.partial(
    pl.kernel,
    out_type=jax.ShapeDtypeStruct((NC, 2, NP), _f32),
    mesh=_mesh,
    scratch_types=[
        pltpu.VMEM((1, C), _i32),
        pltpu.VMEM((1, C), _i32),
        pltpu.VMEM((1, C), _i32),
        pltpu.VMEM((1, C), _i32),
        pltpu.VMEM((C,), _f32),
        pltpu.VMEM((2 * STRIPE,), _f32),
        pltpu.VMEM_SHARED((2 * NP,), _f32),
    ],
)
def _count(ei_hbm, cnt_hbm, i0b, i1b, sib, sob, ones_v, zb, acc):
    c = lax.axis_index("c")
    s = lax.axis_index("s")
    wid = c * NS + s

    def _zinit(k, carry):
        zb[pl.ds(k * 16, 16)] = jnp.zeros((16,), _f32)
        return carry

    lax.fori_loop(0, (2 * STRIPE) // 16, _zinit, 0)
    for k in range(C // 16):
        ones_v[pl.ds(k * 16, 16)] = jnp.ones((16,), _f32)
    pltpu.sync_copy(zb, acc.at[pl.ds(s * (2 * STRIPE), 2 * STRIPE)])
    plsc.subcore_barrier()

    base = wid * EPT_CNT

    def _chunk(j, carry):
        off = base + j * C
        pltpu.sync_copy(ei_hbm.at[pl.ds(off, C)], i0b.at[0])
        pltpu.sync_copy(ei_hbm.at[pl.ds(E + off, C)], i1b.at[0])
        for k in range(C // 16):
            i0 = i0b[0, pl.ds(k * 16, 16)]
            i1 = i1b[0, pl.ds(k * 16, 16)]
            m = i0 == i1
            sib[0, pl.ds(k * 16, 16)] = jnp.where(m, DUMMY, i0)
            sob[0, pl.ds(k * 16, 16)] = jnp.where(m, NP + DUMMY, NP + i1)
        pltpu.sync_copy(ones_v, acc.at[sib.at[0]], add=True)
        pltpu.sync_copy(ones_v, acc.at[sob.at[0]], add=True)
        return carry

    lax.fori_loop(0, NCH_CNT, _chunk, 0)
    plsc.subcore_barrier()
    pltpu.sync_copy(acc.at[pl.ds(s * STRIPE, STRIPE)],
                    cnt_hbm.at[c, 0, pl.ds(s * STRIPE, STRIPE)])
    pltpu.sync_copy(acc.at[pl.ds(NP + s * STRIPE, STRIPE)],
                    cnt_hbm.at[c, 1, pl.ds(s * STRIPE, STRIPE)])


EPT = E // (NC * NS)
NCH = EPT // C
ZROWS = 40


@functools.partial(
    pl.kernel,
    out_type=(jax.ShapeDtypeStruct((NC, NP, H), _f32),
              jax.ShapeDtypeStruct((NC, NP, H), _f32)),
    mesh=_mesh,
    scratch_types=[
        pltpu.VMEM((1, C), _i32),
        pltpu.VMEM((1, C), _i32),
        pltpu.VMEM((1, C), _i32),
        pltpu.VMEM((1, C), _i32),
        pltpu.VMEM((C, H), _f32),
        pltpu.VMEM((C, H), _f32),
        pltpu.VMEM((ZROWS, H), _f32),
        pltpu.VMEM_SHARED((NP, H), _f32),
        pltpu.VMEM_SHARED((NP, H), _f32),
        pltpu.SemaphoreType.DMA,
        pltpu.SemaphoreType.DMA,
    ],
    compiler_params=pltpu.CompilerParams(use_tc_tiling_on_sc=False),
)
def _edge(ei_hbm, xfs_hbm, xbs_hbm, outf_hbm, outb_hbm,
          i0b, i1b, sfb, sbb, gfb, gbb, zb, accf, accb, semf, semb):
    c = lax.axis_index("c")
    s = lax.axis_index("s")

    def _zinit(k, carry):
        zb[k // 4, pl.ds((k % 4) * 16, 16)] = jnp.zeros((16,), _f32)
        return carry

    lax.fori_loop(0, ZROWS * 4, _zinit, 0)
    for t in range(STRIPE // ZROWS):
        pltpu.sync_copy(zb, accf.at[pl.ds(s * STRIPE + t * ZROWS, ZROWS)])
        pltpu.sync_copy(zb, accb.at[pl.ds(s * STRIPE + t * ZROWS, ZROWS)])
    plsc.subcore_barrier()

    base = (c * NS + s) * EPT

    def _chunk(j, carry):
        off = base + j * C
        pltpu.sync_copy(ei_hbm.at[pl.ds(off, C)], i0b.at[0])
        pltpu.sync_copy(ei_hbm.at[pl.ds(E + off, C)], i1b.at[0])
        cpf = pltpu.async_copy(xfs_hbm.at[i1b.at[0]], gfb, semf)
        cpb = pltpu.async_copy(xbs_hbm.at[i0b.at[0]], gbb, semb)
        for k in range(C // 16):
            i0 = i0b[0, pl.ds(k * 16, 16)]
            i1 = i1b[0, pl.ds(k * 16, 16)]
            m = i0 == i1
            sfb[0, pl.ds(k * 16, 16)] = jnp.where(m, DUMMY, i0)
            sbb[0, pl.ds(k * 16, 16)] = jnp.where(m, DUMMY, i1)
        cpf.wait()
        cpb.wait()
        pltpu.sync_copy(gfb, accf.at[sfb.at[0]], add=True)
        pltpu.sync_copy(gbb, accb.at[sbb.at[0]], add=True)
        return carry

    lax.fori_loop(0, NCH, _chunk, 0)
    plsc.subcore_barrier()
    pltpu.sync_copy(accf.at[pl.ds(s * STRIPE, STRIPE)],
                    outf_hbm.at[c].at[pl.ds(s * STRIPE, STRIPE)])
    pltpu.sync_copy(accb.at[pl.ds(s * STRIPE, STRIPE)],
                    outb_hbm.at[c].at[pl.ds(s * STRIPE, STRIPE)])


R = 1280
_GRID = NP // R


def _proj_body(x_ref, wf_ref, wb_ref, cnt_ref, xfs_ref, xbs_ref):
    xblk = x_ref[...]
    xf = lax.dot_general(xblk, wf_ref[...], (((1,), (1,)), ((), ())),
                         preferred_element_type=_f32)
    xb = lax.dot_general(xblk, wb_ref[...], (((1,), (1,)), ((), ())),
                         preferred_element_type=_f32)
    inv_in = lax.rsqrt(cnt_ref[:, 0:1] + cnt_ref[:, 2:3] + 1.0)
    inv_out = lax.rsqrt(cnt_ref[:, 1:2] + cnt_ref[:, 3:4] + 1.0)
    xfs_ref[...] = xf * inv_out
    xbs_ref[...] = xb * inv_in


def _proj(x_pad, W_f, W_b, cnt4):
    return pl.pallas_call(
        _proj_body,
        grid=(_GRID,),
        in_specs=[
            pl.BlockSpec((R, D), lambda i: (i, 0)),
            pl.BlockSpec((H, D), lambda i: (0, 0)),
            pl.BlockSpec((H, D), lambda i: (0, 0)),
            pl.BlockSpec((R, 4), lambda i: (i, 0)),
        ],
        out_specs=[
            pl.BlockSpec((R, H), lambda i: (i, 0)),
            pl.BlockSpec((R, H), lambda i: (i, 0)),
        ],
        out_shape=[jax.ShapeDtypeStruct((NP, H), _f32),
                   jax.ShapeDtypeStruct((NP, H), _f32)],
    )(x_pad, W_f, W_b, cnt4)


def _final_body(aggf_ref, aggb_ref, xfs_ref, xbs_ref, cnt_ref, bias_ref, y_ref):
    inv_in = lax.rsqrt(cnt_ref[:, 0:1] + cnt_ref[:, 2:3] + 1.0)
    inv_out = lax.rsqrt(cnt_ref[:, 1:2] + cnt_ref[:, 3:4] + 1.0)
    yf = inv_in * (aggf_ref[0] + aggf_ref[1] + xfs_ref[...]) + bias_ref[0:1, 0:H]
    yb = inv_out * (aggb_ref[0] + aggb_ref[1] + xbs_ref[...]) + bias_ref[0:1, H:D]
    y_ref[:, 0:H] = yf
    y_ref[:, H:D] = yb


def _final(aggf, aggb, xfs, xbs, cnt4, bias2):
    return pl.pallas_call(
        _final_body,
        grid=(_GRID,),
        in_specs=[
            pl.BlockSpec((NC, R, H), lambda i: (0, i, 0)),
            pl.BlockSpec((NC, R, H), lambda i: (0, i, 0)),
            pl.BlockSpec((R, H), lambda i: (i, 0)),
            pl.BlockSpec((R, H), lambda i: (i, 0)),
            pl.BlockSpec((R, 4), lambda i: (i, 0)),
            pl.BlockSpec((1, D), lambda i: (0, 0)),
        ],
        out_specs=pl.BlockSpec((R, D), lambda i: (i, 0)),
        out_shape=jax.ShapeDtypeStruct((NP, D), _f32),
    )(aggf, aggb, xfs, xbs, cnt4, bias2)


def kernel(x, edge_index, W_f, W_b, bias):
    x_pad = jnp.pad(x, ((0, NP - N), (0, 0)))
    ei_flat = edge_index.reshape(2 * E)
    cnt = _count(ei_flat)
    cnt4 = cnt.transpose(2, 0, 1).reshape(NP, 4)
    xfs, xbs = _proj(x_pad, W_f, W_b, cnt4)
    aggf, aggb = _edge(ei_flat, xfs, xbs)
    y = _final(aggf, aggb, xfs, xbs, cnt4, bias.reshape(1, D))
    return y[:N]


# speedup vs baseline: 38.7411x; 1.9402x over previous
import functools

import jax
import jax.numpy as jnp
from jax import lax
from jax.experimental import pallas as pl
from jax.experimental.pallas import tpu as pltpu
from jax.experimental.pallas import tpu_sc as plsc

N = 10000
NP = 10240
E = 320000
D = 128
H = 64
DUMMY = 10200

NC = 2
NS = 16
C = 80
NBUF = 5
STRIPE = NP // NS

EPT = E // (NC * NS)
NCH = EPT // C
NGRP = NCH // NBUF
EBUF = 4
EGRP = NCH // EBUF
EREM = NCH - EGRP * EBUF

_mesh = plsc.VectorSubcoreMesh(core_axis_name="c", subcore_axis_name="s")

_f32 = jnp.float32
_i32 = jnp.int32


@functools.partial(
    pl.kernel,
    out_type=jax.ShapeDtypeStruct((NC, 2, NP), _f32),
    mesh=_mesh,
    scratch_types=[
        pltpu.VMEM((NBUF, C), _i32),
        pltpu.VMEM((NBUF, C), _i32),
        pltpu.VMEM((NBUF, C), _i32),
        pltpu.VMEM((NBUF, C), _i32),
        pltpu.VMEM((C,), _f32),
        pltpu.VMEM((2 * STRIPE,), _f32),
        pltpu.VMEM_SHARED((2 * NP,), _f32),
        pltpu.SemaphoreType.DMA,
        pltpu.SemaphoreType.DMA,
    ],
)
def _count(ei_hbm, cnt_hbm, i0b, i1b, sib, sob, ones_v, zb, acc, semi, sems):
    c = lax.axis_index("c")
    s = lax.axis_index("s")
    wid = c * NS + s

    def _zinit(k, carry):
        zb[pl.ds(k * 16, 16)] = jnp.zeros((16,), _f32)
        return carry

    lax.fori_loop(0, (2 * STRIPE) // 16, _zinit, 0)
    for k in range(C // 16):
        ones_v[pl.ds(k * 16, 16)] = jnp.ones((16,), _f32)
    pltpu.sync_copy(zb, acc.at[pl.ds(s * (2 * STRIPE), 2 * STRIPE)])
    plsc.subcore_barrier()

    base = wid * EPT

    def _group(g, carry):
        off0 = base + g * (NBUF * C)
        cps = []
        for b in range(NBUF):
            off = off0 + b * C
            cps.append(pltpu.async_copy(ei_hbm.at[pl.ds(off, C)],
                                        i0b.at[b], semi))
            cps.append(pltpu.async_copy(ei_hbm.at[pl.ds(E + off, C)],
                                        i1b.at[b], semi))
        for cp in cps:
            cp.wait()
        for b in range(NBUF):
            for k in range(C // 16):
                i0 = i0b[b, pl.ds(k * 16, 16)]
                i1 = i1b[b, pl.ds(k * 16, 16)]
                m = i0 == i1
                sib[b, pl.ds(k * 16, 16)] = jnp.where(m, DUMMY, i0)
                sob[b, pl.ds(k * 16, 16)] = jnp.where(m, NP + DUMMY, NP + i1)
        sps = []
        for b in range(NBUF):
            sps.append(pltpu.async_copy(ones_v, acc.at[sib.at[b]], sems,
                                        add=True))
            sps.append(pltpu.async_copy(ones_v, acc.at[sob.at[b]], sems,
                                        add=True))
        for cp in sps:
            cp.wait()
        return carry

    lax.fori_loop(0, NGRP, _group, 0)
    plsc.subcore_barrier()
    pltpu.sync_copy(acc.at[pl.ds(s * STRIPE, STRIPE)],
                    cnt_hbm.at[c, 0, pl.ds(s * STRIPE, STRIPE)])
    pltpu.sync_copy(acc.at[pl.ds(NP + s * STRIPE, STRIPE)],
                    cnt_hbm.at[c, 1, pl.ds(s * STRIPE, STRIPE)])


@functools.partial(
    pl.kernel,
    out_type=(jax.ShapeDtypeStruct((NC, NP, H), _f32),
              jax.ShapeDtypeStruct((NC, NP, H), _f32)),
    mesh=_mesh,
    scratch_types=[
        pltpu.VMEM((EBUF, C), _i32),
        pltpu.VMEM((EBUF, C), _i32),
        pltpu.VMEM((EBUF, C), _i32),
        pltpu.VMEM((EBUF, C), _i32),
        pltpu.VMEM((EBUF, C, H), _f32),
        pltpu.VMEM((EBUF, C, H), _f32),
        pltpu.VMEM_SHARED((NP, H), _f32),
        pltpu.VMEM_SHARED((NP, H), _f32),
        pltpu.SemaphoreType.DMA,
        pltpu.SemaphoreType.DMA,
        pltpu.SemaphoreType.DMA,
    ],
    compiler_params=pltpu.CompilerParams(use_tc_tiling_on_sc=False),
)
def _edge(ei_hbm, xfs_hbm, xbs_hbm, outf_hbm, outb_hbm,
          i0b, i1b, sfb, sbb, gfb, gbb, accf, accb, semi, semg, sems):
    c = lax.axis_index("c")
    s = lax.axis_index("s")

    def _zinit(k, carry):
        gfb[0, k // 4, pl.ds((k % 4) * 16, 16)] = jnp.zeros((16,), _f32)
        return carry

    lax.fori_loop(0, C * 4, _zinit, 0)
    for t in range(STRIPE // C):
        pltpu.sync_copy(gfb.at[0], accf.at[pl.ds(s * STRIPE + t * C, C)])
        pltpu.sync_copy(gfb.at[0], accb.at[pl.ds(s * STRIPE + t * C, C)])
    plsc.subcore_barrier()

    base = (c * NS + s) * EPT

    def _do_chunks(off0, nbuf):
        cps = []
        for b in range(nbuf):
            off = off0 + b * C
            cps.append(pltpu.async_copy(ei_hbm.at[pl.ds(off, C)],
                                        i0b.at[b], semi))
            cps.append(pltpu.async_copy(ei_hbm.at[pl.ds(E + off, C)],
                                        i1b.at[b], semi))
        for cp in cps:
            cp.wait()
        gps = []
        for b in range(nbuf):
            gps.append(pltpu.async_copy(xfs_hbm.at[i1b.at[b]], gfb.at[b],
                                        semg))
            gps.append(pltpu.async_copy(xbs_hbm.at[i0b.at[b]], gbb.at[b],
                                        semg))
        for b in range(nbuf):
            for k in range(C // 16):
                i0 = i0b[b, pl.ds(k * 16, 16)]
                i1 = i1b[b, pl.ds(k * 16, 16)]
                m = i0 == i1
                sfb[b, pl.ds(k * 16, 16)] = jnp.where(m, DUMMY, i0)
                sbb[b, pl.ds(k * 16, 16)] = jnp.where(m, DUMMY, i1)
        for cp in gps:
            cp.wait()
        sps = []
        for b in range(nbuf):
            sps.append(pltpu.async_copy(gfb.at[b], accf.at[sfb.at[b]], sems,
                                        add=True))
            sps.append(pltpu.async_copy(gbb.at[b], accb.at[sbb.at[b]], sems,
                                        add=True))
        for cp in sps:
            cp.wait()

    def _group(g, carry):
        _do_chunks(base + g * (EBUF * C), EBUF)
        return carry

    lax.fori_loop(0, EGRP, _group, 0)
    if EREM:
        _do_chunks(base + EGRP * (EBUF * C), EREM)
    plsc.subcore_barrier()
    pltpu.sync_copy(accf.at[pl.ds(s * STRIPE, STRIPE)],
                    outf_hbm.at[c].at[pl.ds(s * STRIPE, STRIPE)])
    pltpu.sync_copy(accb.at[pl.ds(s * STRIPE, STRIPE)],
                    outb_hbm.at[c].at[pl.ds(s * STRIPE, STRIPE)])


R = 1280
_GRID = NP // R


def _proj_body(x_ref, wf_ref, wb_ref, cnt_ref, xfs_ref, xbs_ref):
    xblk = x_ref[...]
    xf = lax.dot_general(xblk, wf_ref[...], (((1,), (1,)), ((), ())),
                         preferred_element_type=_f32)
    xb = lax.dot_general(xblk, wb_ref[...], (((1,), (1,)), ((), ())),
                         preferred_element_type=_f32)
    inv_in = lax.rsqrt(cnt_ref[:, 0:1] + cnt_ref[:, 2:3] + 1.0)
    inv_out = lax.rsqrt(cnt_ref[:, 1:2] + cnt_ref[:, 3:4] + 1.0)
    xfs_ref[...] = xf * inv_out
    xbs_ref[...] = xb * inv_in


def _proj(x_pad, W_f, W_b, cnt4):
    return pl.pallas_call(
        _proj_body,
        grid=(_GRID,),
        in_specs=[
            pl.BlockSpec((R, D), lambda i: (i, 0)),
            pl.BlockSpec((H, D), lambda i: (0, 0)),
            pl.BlockSpec((H, D), lambda i: (0, 0)),
            pl.BlockSpec((R, 4), lambda i: (i, 0)),
        ],
        out_specs=[
            pl.BlockSpec((R, H), lambda i: (i, 0)),
            pl.BlockSpec((R, H), lambda i: (i, 0)),
        ],
        out_shape=[jax.ShapeDtypeStruct((NP, H), _f32),
                   jax.ShapeDtypeStruct((NP, H), _f32)],
    )(x_pad, W_f, W_b, cnt4)


def _final_body(aggf_ref, aggb_ref, xfs_ref, xbs_ref, cnt_ref, bias_ref, y_ref):
    inv_in = lax.rsqrt(cnt_ref[:, 0:1] + cnt_ref[:, 2:3] + 1.0)
    inv_out = lax.rsqrt(cnt_ref[:, 1:2] + cnt_ref[:, 3:4] + 1.0)
    yf = inv_in * (aggf_ref[0] + aggf_ref[1] + xfs_ref[...]) + bias_ref[0:1, 0:H]
    yb = inv_out * (aggb_ref[0] + aggb_ref[1] + xbs_ref[...]) + bias_ref[0:1, H:D]
    y_ref[:, 0:H] = yf
    y_ref[:, H:D] = yb


def _final(aggf, aggb, xfs, xbs, cnt4, bias2):
    return pl.pallas_call(
        _final_body,
        grid=(_GRID,),
        in_specs=[
            pl.BlockSpec((NC, R, H), lambda i: (0, i, 0)),
            pl.BlockSpec((NC, R, H), lambda i: (0, i, 0)),
            pl.BlockSpec((R, H), lambda i: (i, 0)),
            pl.BlockSpec((R, H), lambda i: (i, 0)),
            pl.BlockSpec((R, 4), lambda i: (i, 0)),
            pl.BlockSpec((1, D), lambda i: (0, 0)),
        ],
        out_specs=pl.BlockSpec((R, D), lambda i: (i, 0)),
        out_shape=jax.ShapeDtypeStruct((NP, D), _f32),
    )(aggf, aggb, xfs, xbs, cnt4, bias2)


def kernel(x, edge_index, W_f, W_b, bias):
    x_pad = jnp.pad(x, ((0, NP - N), (0, 0)))
    ei_flat = edge_index.reshape(2 * E)
    cnt = _count(ei_flat)
    cnt4 = cnt.transpose(2, 0, 1).reshape(NP, 4)
    xfs, xbs = _proj(x_pad, W_f, W_b, cnt4)
    aggf, aggb = _edge(ei_flat, xfs, xbs)
    y = _final(aggf, aggb, xfs, xbs, cnt4, bias.reshape(1, D))
    return y[:N]
